# Initial kernel scaffold; baseline (speedup 1.0000x reference)
import jax
import jax.numpy as jnp
from jax.experimental import pallas as pl


def kernel(x, edge_index, W_pre, b_pre, W_post, b_post, W_mix, b_mix):
    raise NotImplementedError("write your pallas kernel here")


# speedup vs baseline: 3.5646x; 3.5646x over previous
import jax
import jax.numpy as jnp
import numpy as np
from jax import lax
from jax.experimental import pallas as pl
from jax.experimental.pallas import tpu as pltpu
from jax.experimental.pallas import tpu_sc as plsc

N = 10000
E = 320000
D = 128
TOWERS = 4
DT = 32
AVG_D_LOG = float(np.log(5.0))
EPS = 1e-5

NW = 32
R = 160
NRANGES = 64
NP = NRANGES * R
C = 1280
NCH = E // C
G16 = C // 16
K = 64
OCAP = 1376


def _sc_body(src_hbm, dst_hbm, a_hbm, b_hbm,
             sum_o, sq_o, mx_o, mn_o, deg_o,
             dstb, srcb, own_sr, own_dst, stg, arows, brows,
             acc_sum, acc_sq, acc_mx, acc_mn, acc_deg, sem_a, sem_b):
  wid = lax.axis_index("c") * 16 + lax.axis_index("s")
  zeros16 = jnp.zeros((16,), jnp.float32)
  zeros16i = jnp.zeros((16,), jnp.int32)
  ones16i = jnp.full((16,), 1, jnp.int32)
  ones16f = jnp.full((16,), 1.0, jnp.float32)
  big16 = jnp.full((16,), 3.4e38, jnp.float32)

  stg[pl.ds(0, 16)] = zeros16i

  for g in range(2):
    rng = wid + 32 * g
    base = rng * R

    def zrow(i, _):
      for j in range(8):
        sl = pl.ds(j * 16, 16)
        acc_sum[i, sl] = zeros16
        acc_sq[i, sl] = zeros16
        acc_mx[i, sl] = zeros16
        acc_mn[i, sl] = big16
      return 0
    lax.fori_loop(0, R, zrow, 0)
    def zdeg(q, _):
      acc_deg[q, :] = zeros16
      return 0
    lax.fori_loop(0, R, zdeg, 0)
    for q in range(8):
      own_sr[pl.ds(q * 16, 16)] = zeros16i
      own_dst[pl.ds(q * 16, 16)] = zeros16i

    def accumulate(i, boff):
      dloc = own_dst[pl.ds(boff + i, 16)][0] - base
      acc_deg[dloc, :] = acc_deg[dloc, :] + ones16f
      for j in range(8):
        sl = pl.ds(j * 16, 16)
        e = jnp.maximum(arows[i, sl] + brows[i, sl], 0.0)
        acc_sum[dloc, sl] = acc_sum[dloc, sl] + e
        acc_sq[dloc, sl] = acc_sq[dloc, sl] + e * e
        acc_mx[dloc, sl] = jnp.maximum(acc_mx[dloc, sl], e)
        acc_mn[dloc, sl] = jnp.minimum(acc_mn[dloc, sl], e)
      return boff

    def chunk_body(c, wp):
      pltpu.sync_copy(dst_hbm.at[pl.ds(c * C, C)], dstb)
      pltpu.sync_copy(src_hbm.at[pl.ds(c * C, C)], srcb)

      def grp(gq, wp):
        sl = pl.ds(gq * 16, 16)
        d16 = dstb[sl]
        msk = (d16 >= base) & (d16 < base + R)
        m0 = jnp.where(msk, 1, 0)
        stg[pl.ds(16, 16)] = m0
        m = m0 + stg[pl.ds(15, 16)]
        stg[pl.ds(16, 16)] = m
        m = m + stg[pl.ds(14, 16)]
        stg[pl.ds(16, 16)] = m
        m = m + stg[pl.ds(12, 16)]
        stg[pl.ds(16, 16)] = m
        m = m + stg[pl.ds(8, 16)]
        cnt = m[15]
        @pl.when(cnt > 0)
        def _():
          pos16 = wp + m - m0
          s16 = srcb[sl]
          for l in range(16):
            p = pos16[l]
            own_sr[pl.ds(p, 16)] = ones16i * s16[l]
            own_dst[pl.ds(p, 16)] = ones16i * d16[l]
        return wp + cnt
      wp = lax.fori_loop(0, G16, grp, wp)

      nb = wp // K

      def blk(bi, _):
        boff = bi * K
        cp_a = pltpu.async_copy(a_hbm.at[own_sr.at[pl.ds(boff, K)]],
                                arows, sem_a)
        cp_b = pltpu.async_copy(b_hbm.at[own_dst.at[pl.ds(boff, K)]],
                                brows, sem_b)
        cp_a.wait()
        cp_b.wait()
        lax.fori_loop(0, K, accumulate, boff)
        return 0
      lax.fori_loop(0, nb, blk, 0)

      rem = wp - nb * K

      def mv(q, _):
        s1 = pl.ds(nb * K + q * 16, 16)
        s2 = pl.ds(q * 16, 16)
        v1 = own_sr[s1]
        own_sr[s2] = v1
        v2 = own_dst[s1]
        own_dst[s2] = v2
        return 0
      lax.fori_loop(0, (rem + 15) // 16, mv, 0)
      return rem

    wp = lax.fori_loop(0, NCH, chunk_body, jnp.int32(0))

    @pl.when(wp > 0)
    def _():
      cp_a = pltpu.async_copy(a_hbm.at[own_sr.at[pl.ds(0, K)]], arows, sem_a)
      cp_b = pltpu.async_copy(b_hbm.at[own_dst.at[pl.ds(0, K)]], brows, sem_b)
      cp_a.wait()
      cp_b.wait()
      lax.fori_loop(0, wp, accumulate, 0)

    osl = pl.ds(base, R)
    pltpu.sync_copy(acc_sum, sum_o.at[osl])
    pltpu.sync_copy(acc_sq, sq_o.at[osl])
    pltpu.sync_copy(acc_mx, mx_o.at[osl])
    pltpu.sync_copy(acc_mn, mn_o.at[osl])
    pltpu.sync_copy(acc_deg, deg_o.at[osl])


def _sc_aggregate(src, dst, a, b):
  mesh = plsc.VectorSubcoreMesh(core_axis_name="c", subcore_axis_name="s",
                                num_cores=2, num_subcores=16)
  f32 = jnp.float32
  out_type = [jax.ShapeDtypeStruct((NP, D), f32) for _ in range(4)]
  out_type.append(jax.ShapeDtypeStruct((NP, 16), f32))
  scratch = [
      pltpu.VMEM((C,), jnp.int32),
      pltpu.VMEM((C,), jnp.int32),
      pltpu.VMEM((OCAP,), jnp.int32),
      pltpu.VMEM((OCAP,), jnp.int32),
      pltpu.VMEM((48,), jnp.int32),
      pltpu.VMEM((K, D), f32),
      pltpu.VMEM((K, D), f32),
      pltpu.VMEM((R, D), f32),
      pltpu.VMEM((R, D), f32),
      pltpu.VMEM((R, D), f32),
      pltpu.VMEM((R, D), f32),
      pltpu.VMEM((R, 16), f32),
      pltpu.SemaphoreType.DMA,
      pltpu.SemaphoreType.DMA,
  ]
  fn = pl.kernel(_sc_body, out_type=out_type, mesh=mesh,
                 scratch_types=scratch)
  return fn(src, dst, a, b)


def _pre_body(x_ref, w1_ref, w2_ref, bias_ref, a_ref, b_ref):
  xb = x_ref[...]
  a_ref[...] = jnp.dot(xb, w1_ref[...], preferred_element_type=jnp.float32)
  b_ref[...] = (jnp.dot(xb, w2_ref[...], preferred_element_type=jnp.float32)
                + bias_ref[0:1, :])


def _pre(xp, w1bd, w2bd, bias2d):
  blk = 1024
  grid = (NP // blk,)
  return pl.pallas_call(
      _pre_body,
      grid=grid,
      in_specs=[
          pl.BlockSpec((blk, D), lambda i: (i, 0)),
          pl.BlockSpec((D, D), lambda i: (0, 0)),
          pl.BlockSpec((D, D), lambda i: (0, 0)),
          pl.BlockSpec((8, D), lambda i: (0, 0)),
      ],
      out_specs=[
          pl.BlockSpec((blk, D), lambda i: (i, 0)),
          pl.BlockSpec((blk, D), lambda i: (i, 0)),
      ],
      out_shape=[jax.ShapeDtypeStruct((NP, D), jnp.float32)] * 2,
  )(xp, w1bd, w2bd, bias2d)


def _post_body(x_ref, sum_ref, sq_ref, mx_ref, mn_ref, deg_ref,
               whbd_ref, dmats_ref, wmix_ref, bpost_ref, bmix_ref, o_ref):
  x = x_ref[...]
  deg = deg_ref[...]
  degc = jnp.maximum(deg, 1.0)
  inv = 1.0 / degc
  logd = jnp.log(degc + 1.0)
  mean = sum_ref[...] * inv
  sqm = sq_ref[...] * inv
  std = jnp.sqrt(jnp.maximum(sqm - mean * mean, 0.0) + EPS)
  has = deg > 0.0
  mx = mx_ref[...]
  mn = jnp.where(has, mn_ref[...], 0.0)
  amp = logd * (1.0 / AVG_D_LOG)
  att = AVG_D_LOG / logd

  def gmat(s):
    acc = jnp.dot(mean, dmats_ref[s * 4 + 0],
                  preferred_element_type=jnp.float32)
    acc += jnp.dot(mx, dmats_ref[s * 4 + 1], preferred_element_type=jnp.float32)
    acc += jnp.dot(mn, dmats_ref[s * 4 + 2], preferred_element_type=jnp.float32)
    acc += jnp.dot(std, dmats_ref[s * 4 + 3],
                   preferred_element_type=jnp.float32)
    return acc

  hcat = (jnp.dot(x, whbd_ref[...], preferred_element_type=jnp.float32)
          + gmat(0) + amp * gmat(1) + att * gmat(2) + bpost_ref[0:1, :])
  ht = jnp.maximum(hcat, 0.0)
  y = jnp.dot(ht, wmix_ref[...], preferred_element_type=jnp.float32) \
      + bmix_ref[0:1, :]
  o_ref[...] = x + jnp.where(y > 0.0, y, 0.01 * y)


def _post(x, sums, sq, mx, mn, deg2d, whbd, dmats, wmix, bpost2d, bmix2d):
  blk = 1000
  grid = (N // blk,)
  node_spec = pl.BlockSpec((blk, D), lambda i: (i, 0))
  full = lambda shape: pl.BlockSpec(shape, lambda i: tuple(0 for _ in shape))
  return pl.pallas_call(
      _post_body,
      grid=grid,
      in_specs=[
          node_spec, node_spec, node_spec, node_spec, node_spec,
          pl.BlockSpec((blk, 1), lambda i: (i, 0)),
          full((D, D)),
          full((12, D, D)),
          full((D, D)),
          full((8, D)),
          full((8, D)),
      ],
      out_specs=node_spec,
      out_shape=jax.ShapeDtypeStruct((N, D), jnp.float32),
  )(x, sums, sq, mx, mn, deg2d, whbd, dmats, wmix, bpost2d, bmix2d)


def _block_diag(blocks):
  z = jnp.zeros((D, D), jnp.float32)
  for t in range(TOWERS):
    z = z.at[t * DT:(t + 1) * DT, t * DT:(t + 1) * DT].set(blocks[t])
  return z


def kernel(x, edge_index, W_pre, b_pre, W_post, b_post, W_mix, b_mix):
  src = edge_index[0]
  dst = edge_index[1]

  w1bd = _block_diag(W_pre[:, :DT, :])
  w2bd = _block_diag(W_pre[:, DT:, :])
  bias2d = jnp.broadcast_to(b_pre.reshape(1, D), (8, D))

  xp = jnp.pad(x, ((0, NP - N), (0, 0)))
  a, b = _pre(xp, w1bd, w2bd, bias2d)

  sums, sq, mx, mn, deg = _sc_aggregate(src, dst, a, b)

  whbd = _block_diag(W_post[:, :DT, :])
  dmats = []
  for s in range(3):
    for p in range(4):
      lo = DT + s * D + p * DT
      dmats.append(_block_diag(W_post[:, lo:lo + DT, :]))
  dmats = jnp.stack(dmats)
  bpost2d = jnp.broadcast_to(b_post.reshape(1, D), (8, D))
  bmix2d = jnp.broadcast_to(b_mix.reshape(1, D), (8, D))

  out = _post(x, sums[:N], sq[:N], mx[:N], mn[:N],
              deg[:N, 0].reshape(N, 1), whbd, dmats, W_mix, bpost2d, bmix2d)
  return out


# speedup vs baseline: 4.2788x; 1.2004x over previous
import jax
import jax.numpy as jnp
import numpy as np
from jax import lax
from jax.experimental import pallas as pl
from jax.experimental.pallas import tpu as pltpu
from jax.experimental.pallas import tpu_sc as plsc

N = 10000
E = 320000
D = 128
TOWERS = 4
DT = 32
AVG_D_LOG = float(np.log(5.0))
EPS = 1e-5

NW = 32
R = 160
NRANGES = 64
NP = NRANGES * R
C = 1280
NCH = E // C
G16 = C // 16
K = 64
OCAP = 1376


def _sc_body(src_hbm, dst_hbm, a_hbm, b_hbm,
             sum_o, sq_o, mx_o, mn_o, deg_o,
             dstb0, dstb1, srcb0, srcb1, own_sr, own_dst, stg, arows, brows,
             acc_sum, acc_sq, acc_mx, acc_mn, acc_deg, sem_a, sem_b,
             sem_d, sem_s):
  dstb = (dstb0, dstb1)
  srcb = (srcb0, srcb1)
  wid = lax.axis_index("c") * 16 + lax.axis_index("s")
  zeros16 = jnp.zeros((16,), jnp.float32)
  zeros16i = jnp.zeros((16,), jnp.int32)
  ones16i = jnp.full((16,), 1, jnp.int32)
  ones16f = jnp.full((16,), 1.0, jnp.float32)
  big16 = jnp.full((16,), 3.4e38, jnp.float32)

  stg[pl.ds(0, 16)] = zeros16i

  for g in range(2):
    rng = wid + 32 * g
    base = rng * R

    pltpu.async_copy(dst_hbm.at[pl.ds(0, C)], dstb[0], sem_d)
    pltpu.async_copy(src_hbm.at[pl.ds(0, C)], srcb[0], sem_s)

    def zrow(i, _):
      for j in range(8):
        sl = pl.ds(j * 16, 16)
        acc_sum[i, sl] = zeros16
        acc_sq[i, sl] = zeros16
        acc_mx[i, sl] = zeros16
        acc_mn[i, sl] = big16
      return 0
    lax.fori_loop(0, R, zrow, 0)
    def zdeg(q, _):
      acc_deg[q, :] = zeros16
      return 0
    lax.fori_loop(0, R, zdeg, 0)
    for q in range(8):
      own_sr[pl.ds(q * 16, 16)] = zeros16i
      own_dst[pl.ds(q * 16, 16)] = zeros16i

    def accumulate(i, boff):
      dloc = own_dst[pl.ds(boff + i, 16)][0] - base
      acc_deg[dloc, :] = acc_deg[dloc, :] + ones16f
      for j in range(8):
        sl = pl.ds(j * 16, 16)
        e = jnp.maximum(arows[i, sl] + brows[i, sl], 0.0)
        acc_sum[dloc, sl] = acc_sum[dloc, sl] + e
        acc_sq[dloc, sl] = acc_sq[dloc, sl] + e * e
        acc_mx[dloc, sl] = jnp.maximum(acc_mx[dloc, sl], e)
        acc_mn[dloc, sl] = jnp.minimum(acc_mn[dloc, sl], e)
      return boff

    def super_body(s, wp):
      for bph in range(2):
        c = s * 2 + bph
        db = dstb[bph]
        sb = srcb[bph]
        pltpu.make_async_copy(dst_hbm.at[pl.ds(0, C)], db, sem_d).wait()
        pltpu.make_async_copy(src_hbm.at[pl.ds(0, C)], sb, sem_s).wait()

        @pl.when(c + 1 < NCH)
        def _():
          pltpu.async_copy(dst_hbm.at[pl.ds((c + 1) * C, C)],
                           dstb[1 - bph], sem_d)
          pltpu.async_copy(src_hbm.at[pl.ds((c + 1) * C, C)],
                           srcb[1 - bph], sem_s)

        def grp(gq, wp):
          sl = pl.ds(gq * 16, 16)
          d16 = db[sl]
          msk = (d16 >= base) & (d16 < base + R)
          m0 = jnp.where(msk, 1, 0)
          stg[pl.ds(16, 16)] = m0
          m = m0 + stg[pl.ds(15, 16)]
          stg[pl.ds(16, 16)] = m
          m = m + stg[pl.ds(14, 16)]
          stg[pl.ds(16, 16)] = m
          m = m + stg[pl.ds(12, 16)]
          stg[pl.ds(16, 16)] = m
          m = m + stg[pl.ds(8, 16)]
          cnt = m[15]
          @pl.when(cnt > 0)
          def _():
            pos16 = wp + m - m0
            s16 = sb[sl]
            for l in range(16):
              p = pos16[l]
              own_sr[pl.ds(p, 16)] = ones16i * s16[l]
              own_dst[pl.ds(p, 16)] = ones16i * d16[l]
          return wp + cnt
        wp = lax.fori_loop(0, G16, grp, wp)

        nb = wp // K

        def blk(bi, _):
          boff = bi * K
          cp_a = pltpu.async_copy(a_hbm.at[own_sr.at[pl.ds(boff, K)]],
                                  arows, sem_a)
          cp_b = pltpu.async_copy(b_hbm.at[own_dst.at[pl.ds(boff, K)]],
                                  brows, sem_b)
          cp_a.wait()
          cp_b.wait()
          lax.fori_loop(0, K, accumulate, boff)
          return 0
        lax.fori_loop(0, nb, blk, 0)

        rem = wp - nb * K

        def mv(q, _):
          s1 = pl.ds(nb * K + q * 16, 16)
          s2 = pl.ds(q * 16, 16)
          v1 = own_sr[s1]
          own_sr[s2] = v1
          v2 = own_dst[s1]
          own_dst[s2] = v2
          return 0
        lax.fori_loop(0, (rem + 15) // 16, mv, 0)
        wp = rem
      return wp

    wp = lax.fori_loop(0, NCH // 2, super_body, jnp.int32(0))

    @pl.when(wp > 0)
    def _():
      cp_a = pltpu.async_copy(a_hbm.at[own_sr.at[pl.ds(0, K)]], arows, sem_a)
      cp_b = pltpu.async_copy(b_hbm.at[own_dst.at[pl.ds(0, K)]], brows, sem_b)
      cp_a.wait()
      cp_b.wait()
      lax.fori_loop(0, wp, accumulate, 0)

    osl = pl.ds(base, R)
    pltpu.sync_copy(acc_sum, sum_o.at[osl])
    pltpu.sync_copy(acc_sq, sq_o.at[osl])
    pltpu.sync_copy(acc_mx, mx_o.at[osl])
    pltpu.sync_copy(acc_mn, mn_o.at[osl])
    pltpu.sync_copy(acc_deg, deg_o.at[osl])


def _sc_aggregate(src, dst, a, b):
  mesh = plsc.VectorSubcoreMesh(core_axis_name="c", subcore_axis_name="s",
                                num_cores=2, num_subcores=16)
  f32 = jnp.float32
  out_type = [jax.ShapeDtypeStruct((NP, D), f32) for _ in range(4)]
  out_type.append(jax.ShapeDtypeStruct((NP, 16), f32))
  scratch = [
      pltpu.VMEM((C,), jnp.int32),
      pltpu.VMEM((C,), jnp.int32),
      pltpu.VMEM((C,), jnp.int32),
      pltpu.VMEM((C,), jnp.int32),
      pltpu.VMEM((OCAP,), jnp.int32),
      pltpu.VMEM((OCAP,), jnp.int32),
      pltpu.VMEM((48,), jnp.int32),
      pltpu.VMEM((K, D), f32),
      pltpu.VMEM((K, D), f32),
      pltpu.VMEM((R, D), f32),
      pltpu.VMEM((R, D), f32),
      pltpu.VMEM((R, D), f32),
      pltpu.VMEM((R, D), f32),
      pltpu.VMEM((R, 16), f32),
      pltpu.SemaphoreType.DMA,
      pltpu.SemaphoreType.DMA,
      pltpu.SemaphoreType.DMA,
      pltpu.SemaphoreType.DMA,
  ]
  fn = pl.kernel(_sc_body, out_type=out_type, mesh=mesh,
                 scratch_types=scratch)
  return fn(src, dst, a, b)


def _pre_body(x_ref, w1_ref, w2_ref, bias_ref, a_ref, b_ref):
  xb = x_ref[...]
  a_ref[...] = jnp.dot(xb, w1_ref[...], preferred_element_type=jnp.float32)
  b_ref[...] = (jnp.dot(xb, w2_ref[...], preferred_element_type=jnp.float32)
                + bias_ref[0:1, :])


def _pre(xp, w1bd, w2bd, bias2d):
  blk = 1024
  grid = (NP // blk,)
  return pl.pallas_call(
      _pre_body,
      grid=grid,
      in_specs=[
          pl.BlockSpec((blk, D), lambda i: (i, 0)),
          pl.BlockSpec((D, D), lambda i: (0, 0)),
          pl.BlockSpec((D, D), lambda i: (0, 0)),
          pl.BlockSpec((8, D), lambda i: (0, 0)),
      ],
      out_specs=[
          pl.BlockSpec((blk, D), lambda i: (i, 0)),
          pl.BlockSpec((blk, D), lambda i: (i, 0)),
      ],
      out_shape=[jax.ShapeDtypeStruct((NP, D), jnp.float32)] * 2,
  )(xp, w1bd, w2bd, bias2d)


def _post_body(x_ref, sum_ref, sq_ref, mx_ref, mn_ref, deg_ref,
               whbd_ref, dmats_ref, wmix_ref, bpost_ref, bmix_ref, o_ref):
  x = x_ref[...]
  deg = deg_ref[...]
  degc = jnp.maximum(deg, 1.0)
  inv = 1.0 / degc
  logd = jnp.log(degc + 1.0)
  mean = sum_ref[...] * inv
  sqm = sq_ref[...] * inv
  std = jnp.sqrt(jnp.maximum(sqm - mean * mean, 0.0) + EPS)
  has = deg > 0.0
  mx = mx_ref[...]
  mn = jnp.where(has, mn_ref[...], 0.0)
  amp = logd * (1.0 / AVG_D_LOG)
  att = AVG_D_LOG / logd

  def gmat(s):
    acc = jnp.dot(mean, dmats_ref[s * 4 + 0],
                  preferred_element_type=jnp.float32)
    acc += jnp.dot(mx, dmats_ref[s * 4 + 1], preferred_element_type=jnp.float32)
    acc += jnp.dot(mn, dmats_ref[s * 4 + 2], preferred_element_type=jnp.float32)
    acc += jnp.dot(std, dmats_ref[s * 4 + 3],
                   preferred_element_type=jnp.float32)
    return acc

  hcat = (jnp.dot(x, whbd_ref[...], preferred_element_type=jnp.float32)
          + gmat(0) + amp * gmat(1) + att * gmat(2) + bpost_ref[0:1, :])
  ht = jnp.maximum(hcat, 0.0)
  y = jnp.dot(ht, wmix_ref[...], preferred_element_type=jnp.float32) \
      + bmix_ref[0:1, :]
  o_ref[...] = x + jnp.where(y > 0.0, y, 0.01 * y)


def _post(x, sums, sq, mx, mn, deg2d, whbd, dmats, wmix, bpost2d, bmix2d):
  blk = 1000
  grid = (N // blk,)
  node_spec = pl.BlockSpec((blk, D), lambda i: (i, 0))
  full = lambda shape: pl.BlockSpec(shape, lambda i: tuple(0 for _ in shape))
  return pl.pallas_call(
      _post_body,
      grid=grid,
      in_specs=[
          node_spec, node_spec, node_spec, node_spec, node_spec,
          pl.BlockSpec((blk, 1), lambda i: (i, 0)),
          full((D, D)),
          full((12, D, D)),
          full((D, D)),
          full((8, D)),
          full((8, D)),
      ],
      out_specs=node_spec,
      out_shape=jax.ShapeDtypeStruct((N, D), jnp.float32),
  )(x, sums, sq, mx, mn, deg2d, whbd, dmats, wmix, bpost2d, bmix2d)


def _block_diag(blocks):
  z = jnp.zeros((D, D), jnp.float32)
  for t in range(TOWERS):
    z = z.at[t * DT:(t + 1) * DT, t * DT:(t + 1) * DT].set(blocks[t])
  return z


def kernel(x, edge_index, W_pre, b_pre, W_post, b_post, W_mix, b_mix):
  src = edge_index[0]
  dst = edge_index[1]

  w1bd = _block_diag(W_pre[:, :DT, :])
  w2bd = _block_diag(W_pre[:, DT:, :])
  bias2d = jnp.broadcast_to(b_pre.reshape(1, D), (8, D))

  xp = jnp.pad(x, ((0, NP - N), (0, 0)))
  a, b = _pre(xp, w1bd, w2bd, bias2d)

  sums, sq, mx, mn, deg = _sc_aggregate(src, dst, a, b)

  whbd = _block_diag(W_post[:, :DT, :])
  dmats = []
  for s in range(3):
    for p in range(4):
      lo = DT + s * D + p * DT
      dmats.append(_block_diag(W_post[:, lo:lo + DT, :]))
  dmats = jnp.stack(dmats)
  bpost2d = jnp.broadcast_to(b_post.reshape(1, D), (8, D))
  bmix2d = jnp.broadcast_to(b_mix.reshape(1, D), (8, D))

  out = _post(x, sums[:N], sq[:N], mx[:N], mn[:N],
              deg[:N, 0].reshape(N, 1), whbd, dmats, W_mix, bpost2d, bmix2d)
  return out


# speedup vs baseline: 4.2863x; 1.0018x over previous
import jax
import jax.numpy as jnp
import numpy as np
from jax import lax
from jax.experimental import pallas as pl
from jax.experimental.pallas import tpu as pltpu
from jax.experimental.pallas import tpu_sc as plsc

N = 10000
E = 320000
D = 128
TOWERS = 4
DT = 32
AVG_D_LOG = float(np.log(5.0))
EPS = 1e-5

NW = 32
R = 160
NRANGES = 64
NP = NRANGES * R
C = 1600
NCH = E // C
G16 = C // 16
K = 64
OCAP = 1696


def _sc_body(src_hbm, dst_hbm, a_hbm, b_hbm,
             sum_o, sq_o, mx_o, mn_o, deg_o,
             dstb0, dstb1, srcb0, srcb1, own_sr, own_dst, stg, arows, brows,
             acc_sum, acc_sq, acc_mx, acc_mn, acc_deg, sem_a, sem_b,
             sem_d, sem_s):
  dstb = (dstb0, dstb1)
  srcb = (srcb0, srcb1)
  wid = lax.axis_index("c") * 16 + lax.axis_index("s")
  zeros16 = jnp.zeros((16,), jnp.float32)
  zeros16i = jnp.zeros((16,), jnp.int32)
  ones16i = jnp.full((16,), 1, jnp.int32)
  ones16f = jnp.full((16,), 1.0, jnp.float32)
  big16 = jnp.full((16,), 3.4e38, jnp.float32)

  stg[pl.ds(0, 16)] = zeros16i

  for g in range(2):
    rng = wid + 32 * g
    base = rng * R

    pltpu.async_copy(dst_hbm.at[pl.ds(0, C)], dstb[0], sem_d)
    pltpu.async_copy(src_hbm.at[pl.ds(0, C)], srcb[0], sem_s)

    def zrow(i, _):
      for j in range(8):
        sl = pl.ds(j * 16, 16)
        acc_sum[i, sl] = zeros16
        acc_sq[i, sl] = zeros16
        acc_mx[i, sl] = zeros16
        acc_mn[i, sl] = big16
      return 0
    lax.fori_loop(0, R, zrow, 0)
    def zdeg(q, _):
      acc_deg[q, :] = zeros16
      return 0
    lax.fori_loop(0, R, zdeg, 0)
    for q in range(8):
      own_sr[pl.ds(q * 16, 16)] = zeros16i
      own_dst[pl.ds(q * 16, 16)] = zeros16i

    def accumulate(i, boff):
      dloc = own_dst[pl.ds(boff + i, 16)][0] - base
      acc_deg[dloc, :] = acc_deg[dloc, :] + ones16f
      for j in range(8):
        sl = pl.ds(j * 16, 16)
        e = jnp.maximum(arows[i, sl] + brows[i, sl], 0.0)
        acc_sum[dloc, sl] = acc_sum[dloc, sl] + e
        acc_sq[dloc, sl] = acc_sq[dloc, sl] + e * e
        acc_mx[dloc, sl] = jnp.maximum(acc_mx[dloc, sl], e)
        acc_mn[dloc, sl] = jnp.minimum(acc_mn[dloc, sl], e)
      return boff

    def super_body(s, wp):
      for bph in range(2):
        c = s * 2 + bph
        db = dstb[bph]
        sb = srcb[bph]
        pltpu.make_async_copy(dst_hbm.at[pl.ds(0, C)], db, sem_d).wait()
        pltpu.make_async_copy(src_hbm.at[pl.ds(0, C)], sb, sem_s).wait()

        @pl.when(c + 1 < NCH)
        def _():
          pltpu.async_copy(dst_hbm.at[pl.ds((c + 1) * C, C)],
                           dstb[1 - bph], sem_d)
          pltpu.async_copy(src_hbm.at[pl.ds((c + 1) * C, C)],
                           srcb[1 - bph], sem_s)

        def grp(gq, wp):
          sl = pl.ds(gq * 16, 16)
          d16 = db[sl]
          msk = (d16 >= base) & (d16 < base + R)
          m0 = jnp.where(msk, 1, 0)
          stg[pl.ds(16, 16)] = m0
          m = m0 + stg[pl.ds(15, 16)]
          stg[pl.ds(16, 16)] = m
          m = m + stg[pl.ds(14, 16)]
          stg[pl.ds(16, 16)] = m
          m = m + stg[pl.ds(12, 16)]
          stg[pl.ds(16, 16)] = m
          m = m + stg[pl.ds(8, 16)]
          cnt = m[15]
          @pl.when(cnt > 0)
          def _():
            pos16 = wp + m - m0
            s16 = sb[sl]
            for l in range(16):
              p = pos16[l]
              own_sr[pl.ds(p, 16)] = ones16i * s16[l]
              own_dst[pl.ds(p, 16)] = ones16i * d16[l]
          return wp + cnt
        wp = lax.fori_loop(0, G16, grp, wp)

        nb = wp // K

        def blk(bi, _):
          boff = bi * K
          cp_a = pltpu.async_copy(a_hbm.at[own_sr.at[pl.ds(boff, K)]],
                                  arows, sem_a)
          cp_b = pltpu.async_copy(b_hbm.at[own_dst.at[pl.ds(boff, K)]],
                                  brows, sem_b)
          cp_a.wait()
          cp_b.wait()
          lax.fori_loop(0, K, accumulate, boff)
          return 0
        lax.fori_loop(0, nb, blk, 0)

        rem = wp - nb * K

        def mv(q, _):
          s1 = pl.ds(nb * K + q * 16, 16)
          s2 = pl.ds(q * 16, 16)
          v1 = own_sr[s1]
          own_sr[s2] = v1
          v2 = own_dst[s1]
          own_dst[s2] = v2
          return 0
        lax.fori_loop(0, (rem + 15) // 16, mv, 0)
        wp = rem
      return wp

    wp = lax.fori_loop(0, NCH // 2, super_body, jnp.int32(0))

    @pl.when(wp > 0)
    def _():
      cp_a = pltpu.async_copy(a_hbm.at[own_sr.at[pl.ds(0, K)]], arows, sem_a)
      cp_b = pltpu.async_copy(b_hbm.at[own_dst.at[pl.ds(0, K)]], brows, sem_b)
      cp_a.wait()
      cp_b.wait()
      lax.fori_loop(0, wp, accumulate, 0)

    osl = pl.ds(base, R)
    pltpu.sync_copy(acc_sum, sum_o.at[osl])
    pltpu.sync_copy(acc_sq, sq_o.at[osl])
    pltpu.sync_copy(acc_mx, mx_o.at[osl])
    pltpu.sync_copy(acc_mn, mn_o.at[osl])
    pltpu.sync_copy(acc_deg, deg_o.at[osl])


def _sc_aggregate(src, dst, a, b):
  mesh = plsc.VectorSubcoreMesh(core_axis_name="c", subcore_axis_name="s",
                                num_cores=2, num_subcores=16)
  f32 = jnp.float32
  out_type = [jax.ShapeDtypeStruct((NP, D), f32) for _ in range(4)]
  out_type.append(jax.ShapeDtypeStruct((NP, 16), f32))
  scratch = [
      pltpu.VMEM((C,), jnp.int32),
      pltpu.VMEM((C,), jnp.int32),
      pltpu.VMEM((C,), jnp.int32),
      pltpu.VMEM((C,), jnp.int32),
      pltpu.VMEM((OCAP,), jnp.int32),
      pltpu.VMEM((OCAP,), jnp.int32),
      pltpu.VMEM((48,), jnp.int32),
      pltpu.VMEM((K, D), f32),
      pltpu.VMEM((K, D), f32),
      pltpu.VMEM((R, D), f32),
      pltpu.VMEM((R, D), f32),
      pltpu.VMEM((R, D), f32),
      pltpu.VMEM((R, D), f32),
      pltpu.VMEM((R, 16), f32),
      pltpu.SemaphoreType.DMA,
      pltpu.SemaphoreType.DMA,
      pltpu.SemaphoreType.DMA,
      pltpu.SemaphoreType.DMA,
  ]
  fn = pl.kernel(_sc_body, out_type=out_type, mesh=mesh,
                 scratch_types=scratch)
  return fn(src, dst, a, b)


def _pre_body(x_ref, w1_ref, w2_ref, bias_ref, a_ref, b_ref):
  xb = x_ref[...]
  a_ref[...] = jnp.dot(xb, w1_ref[...], preferred_element_type=jnp.float32)
  b_ref[...] = (jnp.dot(xb, w2_ref[...], preferred_element_type=jnp.float32)
                + bias_ref[0:1, :])


def _pre(xp, w1bd, w2bd, bias2d):
  blk = 1024
  grid = (NP // blk,)
  return pl.pallas_call(
      _pre_body,
      grid=grid,
      in_specs=[
          pl.BlockSpec((blk, D), lambda i: (i, 0)),
          pl.BlockSpec((D, D), lambda i: (0, 0)),
          pl.BlockSpec((D, D), lambda i: (0, 0)),
          pl.BlockSpec((8, D), lambda i: (0, 0)),
      ],
      out_specs=[
          pl.BlockSpec((blk, D), lambda i: (i, 0)),
          pl.BlockSpec((blk, D), lambda i: (i, 0)),
      ],
      out_shape=[jax.ShapeDtypeStruct((NP, D), jnp.float32)] * 2,
  )(xp, w1bd, w2bd, bias2d)


def _post_body(x_ref, sum_ref, sq_ref, mx_ref, mn_ref, deg_ref,
               whbd_ref, dmats_ref, wmix_ref, bpost_ref, bmix_ref, o_ref):
  x = x_ref[...]
  deg = deg_ref[...]
  degc = jnp.maximum(deg, 1.0)
  inv = 1.0 / degc
  logd = jnp.log(degc + 1.0)
  mean = sum_ref[...] * inv
  sqm = sq_ref[...] * inv
  std = jnp.sqrt(jnp.maximum(sqm - mean * mean, 0.0) + EPS)
  has = deg > 0.0
  mx = mx_ref[...]
  mn = jnp.where(has, mn_ref[...], 0.0)
  amp = logd * (1.0 / AVG_D_LOG)
  att = AVG_D_LOG / logd

  def gmat(s):
    acc = jnp.dot(mean, dmats_ref[s * 4 + 0],
                  preferred_element_type=jnp.float32)
    acc += jnp.dot(mx, dmats_ref[s * 4 + 1], preferred_element_type=jnp.float32)
    acc += jnp.dot(mn, dmats_ref[s * 4 + 2], preferred_element_type=jnp.float32)
    acc += jnp.dot(std, dmats_ref[s * 4 + 3],
                   preferred_element_type=jnp.float32)
    return acc

  hcat = (jnp.dot(x, whbd_ref[...], preferred_element_type=jnp.float32)
          + gmat(0) + amp * gmat(1) + att * gmat(2) + bpost_ref[0:1, :])
  ht = jnp.maximum(hcat, 0.0)
  y = jnp.dot(ht, wmix_ref[...], preferred_element_type=jnp.float32) \
      + bmix_ref[0:1, :]
  o_ref[...] = x + jnp.where(y > 0.0, y, 0.01 * y)


def _post(x, sums, sq, mx, mn, deg2d, whbd, dmats, wmix, bpost2d, bmix2d):
  blk = 1000
  grid = (N // blk,)
  node_spec = pl.BlockSpec((blk, D), lambda i: (i, 0))
  full = lambda shape: pl.BlockSpec(shape, lambda i: tuple(0 for _ in shape))
  return pl.pallas_call(
      _post_body,
      grid=grid,
      in_specs=[
          node_spec, node_spec, node_spec, node_spec, node_spec,
          pl.BlockSpec((blk, 1), lambda i: (i, 0)),
          full((D, D)),
          full((12, D, D)),
          full((D, D)),
          full((8, D)),
          full((8, D)),
      ],
      out_specs=node_spec,
      out_shape=jax.ShapeDtypeStruct((N, D), jnp.float32),
  )(x, sums, sq, mx, mn, deg2d, whbd, dmats, wmix, bpost2d, bmix2d)


def _block_diag(blocks):
  z = jnp.zeros((D, D), jnp.float32)
  for t in range(TOWERS):
    z = z.at[t * DT:(t + 1) * DT, t * DT:(t + 1) * DT].set(blocks[t])
  return z


def kernel(x, edge_index, W_pre, b_pre, W_post, b_post, W_mix, b_mix):
  src = edge_index[0]
  dst = edge_index[1]

  w1bd = _block_diag(W_pre[:, :DT, :])
  w2bd = _block_diag(W_pre[:, DT:, :])
  bias2d = jnp.broadcast_to(b_pre.reshape(1, D), (8, D))

  xp = jnp.pad(x, ((0, NP - N), (0, 0)))
  a, b = _pre(xp, w1bd, w2bd, bias2d)

  sums, sq, mx, mn, deg = _sc_aggregate(src, dst, a, b)

  whbd = _block_diag(W_post[:, :DT, :])
  dmats = []
  for s in range(3):
    for p in range(4):
      lo = DT + s * D + p * DT
      dmats.append(_block_diag(W_post[:, lo:lo + DT, :]))
  dmats = jnp.stack(dmats)
  bpost2d = jnp.broadcast_to(b_post.reshape(1, D), (8, D))
  bmix2d = jnp.broadcast_to(b_mix.reshape(1, D), (8, D))

  out = _post(x, sums[:N], sq[:N], mx[:N], mn[:N],
              deg[:N, 0].reshape(N, 1), whbd, dmats, W_mix, bpost2d, bmix2d)
  return out
